# baseline (speedup 1.0000x reference)
import functools

import jax
import jax.numpy as jnp
from jax import lax
from jax.experimental import pallas as pl
from jax.experimental.pallas import tpu as pltpu
from jax.experimental.pallas import tpu_sc as plsc

_V, _D = 1_000_000, 64
_B, _C, _K = 16384, 20, 20
_NC, _NS = 2, 16
_NW = _NC * _NS
_BPW = _B // _NW
_NB = 16
_NCHUNK = _BPW // _NB
_ROWS = _NB * _C
_NSTREAM = 4
_SROWS = _ROWS // _NSTREAM
_L = 16


def _sc_body(idxu_hbm, idxn_hbm, idxt_hbm, emb_u_hbm, emb_v_hbm,
             pos_hbm, neg_hbm,
             idx_u, idx_n, idx_t, rows_u, rows_n, rows_t,
             pos_buf, neg_buf, sem):
    wid = lax.axis_index("s") * _NC + lax.axis_index("c")
    base = wid * _BPW
    pltpu.sync_copy(idxu_hbm.at[pl.ds(base * _C, _BPW * _C)], idx_u)
    pltpu.sync_copy(idxn_hbm.at[pl.ds(base * _K, _BPW * _K)], idx_n)
    pltpu.sync_copy(idxt_hbm.at[pl.ds(base, _BPW)], idx_t)

    lanes = lax.iota(jnp.int32, _L)

    def _lane_sum(x):
        for s in (8, 4, 2, 1):
            x = x + x.at[(lanes + s) % _L].get(mode="promise_in_bounds")
        return x

    def chunk_body(ci, _):
        off = ci * _ROWS
        cps = []
        for s in range(_NSTREAM):
            cps.append(pltpu.async_copy(
                emb_u_hbm.at[idx_u.at[pl.ds(off + s * _SROWS, _SROWS)]],
                rows_u.at[pl.ds(s * _SROWS, _SROWS), :], sem))
            cps.append(pltpu.async_copy(
                emb_v_hbm.at[idx_n.at[pl.ds(off + s * _SROWS, _SROWS)]],
                rows_n.at[pl.ds(s * _SROWS, _SROWS), :], sem))
        cps.append(pltpu.async_copy(
            emb_v_hbm.at[idx_t.at[pl.ds(ci * _NB, _NB)]], rows_t, sem))
        for cp in cps:
            cp.wait()

        def b_body(j, carry):
            acc_p, acc_n = carry
            rb = j * _C
            su = [jnp.zeros((_L,), jnp.float32) for _ in range(_D // _L)]
            sn = [jnp.zeros((_L,), jnp.float32) for _ in range(_D // _L)]
            for c in range(_C):
                for blk in range(_D // _L):
                    su[blk] = su[blk] + rows_u[rb + c, pl.ds(blk * _L, _L)]
                    sn[blk] = sn[blk] + rows_n[rb + c, pl.ds(blk * _L, _L)]
            pp = jnp.zeros((_L,), jnp.float32)
            nn = jnp.zeros((_L,), jnp.float32)
            for blk in range(_D // _L):
                t_blk = rows_t[j, pl.ds(blk * _L, _L)]
                pp = pp + t_blk * su[blk]
                nn = nn + sn[blk] * su[blk]
            m = lanes == j
            acc_p = jnp.where(m, _lane_sum(pp) * (1.0 / _C), acc_p)
            acc_n = jnp.where(m, _lane_sum(nn) * (1.0 / _C), acc_n)
            return acc_p, acc_n

        acc_p, acc_n = lax.fori_loop(
            0, _NB, b_body,
            (jnp.zeros((_L,), jnp.float32), jnp.zeros((_L,), jnp.float32)))
        pos_buf[pl.ds(ci * _NB, _NB)] = acc_p
        neg_buf[pl.ds(ci * _NB, _NB)] = acc_n
        return 0

    lax.fori_loop(0, _NCHUNK, chunk_body, 0)
    pltpu.sync_copy(pos_buf, pos_hbm.at[pl.ds(base, _BPW)])
    pltpu.sync_copy(neg_buf, neg_hbm.at[pl.ds(base, _BPW)])


@functools.lru_cache(maxsize=None)
def _sc_scores():
    return functools.partial(
        pl.kernel,
        mesh=plsc.VectorSubcoreMesh(core_axis_name="c", subcore_axis_name="s"),
        compiler_params=pltpu.CompilerParams(use_tc_tiling_on_sc=False),
        out_type=[jax.ShapeDtypeStruct((_B,), jnp.float32),
                  jax.ShapeDtypeStruct((_B,), jnp.float32)],
        scratch_types=[
            pltpu.VMEM((_BPW * _C,), jnp.int32),
            pltpu.VMEM((_BPW * _K,), jnp.int32),
            pltpu.VMEM((_BPW,), jnp.int32),
            pltpu.VMEM((_ROWS, _D), jnp.float32),
            pltpu.VMEM((_ROWS, _D), jnp.float32),
            pltpu.VMEM((_NB, _D), jnp.float32),
            pltpu.VMEM((_BPW,), jnp.float32),
            pltpu.VMEM((_BPW,), jnp.float32),
            pltpu.SemaphoreType.DMA,
        ],
    )(_sc_body)


def _finish_body(pos_ref, neg_ref, out_ref):
    p = pos_ref[...]
    n = neg_ref[...]
    lsp = jnp.minimum(p, 0.0) - jnp.log1p(jnp.exp(-jnp.abs(p)))
    lsn = jnp.minimum(-n, 0.0) - jnp.log1p(jnp.exp(-jnp.abs(n)))
    out_ref[0, 0] = -(jnp.sum(lsp) + jnp.sum(lsn)) / _B


def _finish(pos2d, neg2d):
    return pl.pallas_call(
        _finish_body,
        out_shape=jax.ShapeDtypeStruct((1, 1), jnp.float32),
        out_specs=pl.BlockSpec(memory_space=pltpu.SMEM),
    )(pos2d, neg2d)


_TBLK = 512


def _transpose_body(in_ref, out_ref):
    out_ref[...] = in_ref[...].T


def _linearize(table):
    tt = table.T
    return pl.pallas_call(
        _transpose_body,
        grid=(_V // _TBLK,),
        in_specs=[pl.BlockSpec((_D, _TBLK), lambda i: (0, i))],
        out_specs=pl.BlockSpec((_TBLK, _D), lambda i: (i, 0)),
        out_shape=jax.ShapeDtypeStruct((_V, _D), jnp.float32),
    )(tt)


def kernel(inputs, targets, negatives, emb_u, emb_v):
    idxu = inputs.astype(jnp.int32).reshape(-1)
    idxn = negatives.astype(jnp.int32).reshape(-1)
    idxt = targets.astype(jnp.int32)
    pos, neg = _sc_scores()(idxu, idxn, idxt,
                            _linearize(emb_u), _linearize(emb_v))
    res = _finish(pos.reshape(128, 128), neg.reshape(128, 128))
    return res[0, 0]


# speedup vs baseline: 1.7494x; 1.7494x over previous
import functools

import jax
import jax.numpy as jnp
from jax import lax
from jax.experimental import pallas as pl
from jax.experimental.pallas import tpu as pltpu
from jax.experimental.pallas import tpu_sc as plsc

_V, _D = 1_000_000, 64
_B, _C, _K = 16384, 20, 20
_NC, _NS = 2, 16
_NW = _NC * _NS
_BPW = _B // _NW
_NB = 16
_NCHUNK = _BPW // _NB
_ROWS = _NB * _C
_NSTREAM = 4
_SROWS = _ROWS // _NSTREAM
_L = 16


def _sc_body(idxu_hbm, idxn_hbm, idxt_hbm, emb_u_hbm, emb_v_hbm,
             pos_hbm, neg_hbm,
             idx_u, idx_n, idx_t, rows_u, rows_n, rows_t,
             pos_buf, neg_buf, sem):
    wid = lax.axis_index("s") * _NC + lax.axis_index("c")
    base = wid * _BPW
    pltpu.sync_copy(idxu_hbm.at[pl.ds(base * _C, _BPW * _C)], idx_u)
    pltpu.sync_copy(idxn_hbm.at[pl.ds(base * _K, _BPW * _K)], idx_n)
    pltpu.sync_copy(idxt_hbm.at[pl.ds(base, _BPW)], idx_t)

    lanes = lax.iota(jnp.int32, _L)

    def _lane_sum(x):
        for s in (8, 4, 2, 1):
            x = x + x.at[(lanes + s) % _L].get(mode="promise_in_bounds")
        return x

    def chunk_body(ci, _):
        off = ci * _ROWS
        cps = []
        for s in range(_NSTREAM):
            cps.append(pltpu.async_copy(
                emb_u_hbm.at[idx_u.at[pl.ds(off + s * _SROWS, _SROWS)]],
                rows_u.at[pl.ds(s * _SROWS, _SROWS), :], sem))
            cps.append(pltpu.async_copy(
                emb_v_hbm.at[idx_n.at[pl.ds(off + s * _SROWS, _SROWS)]],
                rows_n.at[pl.ds(s * _SROWS, _SROWS), :], sem))
        cps.append(pltpu.async_copy(
            emb_v_hbm.at[idx_t.at[pl.ds(ci * _NB, _NB)]], rows_t, sem))
        for cp in cps:
            cp.wait()

        def b_body(j, carry):
            acc_p, acc_n = carry
            rb = j * _C
            su = [jnp.zeros((_L,), jnp.float32) for _ in range(_D // _L)]
            sn = [jnp.zeros((_L,), jnp.float32) for _ in range(_D // _L)]
            for c in range(_C):
                for blk in range(_D // _L):
                    su[blk] = su[blk] + rows_u[rb + c, pl.ds(blk * _L, _L)]
                    sn[blk] = sn[blk] + rows_n[rb + c, pl.ds(blk * _L, _L)]
            pp = jnp.zeros((_L,), jnp.float32)
            nn = jnp.zeros((_L,), jnp.float32)
            for blk in range(_D // _L):
                t_blk = rows_t[j, pl.ds(blk * _L, _L)]
                pp = pp + t_blk * su[blk]
                nn = nn + sn[blk] * su[blk]
            m = lanes == j
            acc_p = jnp.where(m, _lane_sum(pp) * (1.0 / _C), acc_p)
            acc_n = jnp.where(m, _lane_sum(nn) * (1.0 / _C), acc_n)
            return acc_p, acc_n

        acc_p, acc_n = lax.fori_loop(
            0, _NB, b_body,
            (jnp.zeros((_L,), jnp.float32), jnp.zeros((_L,), jnp.float32)))
        pos_buf[pl.ds(ci * _NB, _NB)] = acc_p
        neg_buf[pl.ds(ci * _NB, _NB)] = acc_n
        return 0

    lax.fori_loop(0, _NCHUNK, chunk_body, 0)
    pltpu.sync_copy(pos_buf, pos_hbm.at[pl.ds(base, _BPW)])
    pltpu.sync_copy(neg_buf, neg_hbm.at[pl.ds(base, _BPW)])


@functools.lru_cache(maxsize=None)
def _sc_scores():
    return functools.partial(
        pl.kernel,
        mesh=plsc.VectorSubcoreMesh(core_axis_name="c", subcore_axis_name="s"),
        compiler_params=pltpu.CompilerParams(use_tc_tiling_on_sc=False),
        out_type=[jax.ShapeDtypeStruct((_B,), jnp.float32),
                  jax.ShapeDtypeStruct((_B,), jnp.float32)],
        scratch_types=[
            pltpu.VMEM((_BPW * _C,), jnp.int32),
            pltpu.VMEM((_BPW * _K,), jnp.int32),
            pltpu.VMEM((_BPW,), jnp.int32),
            pltpu.VMEM((_ROWS, _D), jnp.float32),
            pltpu.VMEM((_ROWS, _D), jnp.float32),
            pltpu.VMEM((_NB, _D), jnp.float32),
            pltpu.VMEM((_BPW,), jnp.float32),
            pltpu.VMEM((_BPW,), jnp.float32),
            pltpu.SemaphoreType.DMA,
        ],
    )(_sc_body)


def _finish_body(pos_ref, neg_ref, out_ref):
    p = pos_ref[...]
    n = neg_ref[...]
    lsp = jnp.minimum(p, 0.0) - jnp.log1p(jnp.exp(-jnp.abs(p)))
    lsn = jnp.minimum(-n, 0.0) - jnp.log1p(jnp.exp(-jnp.abs(n)))
    out_ref[0, 0] = -(jnp.sum(lsp) + jnp.sum(lsn)) / _B


def _finish(pos2d, neg2d):
    return pl.pallas_call(
        _finish_body,
        out_shape=jax.ShapeDtypeStruct((1, 1), jnp.float32),
        out_specs=pl.BlockSpec(memory_space=pltpu.SMEM),
    )(pos2d, neg2d)


_TBLK = 2048


def _transpose_body(in_ref, out_ref):
    out_ref[...] = lax.dot_general(
        in_ref[...], jnp.eye(_D, dtype=jnp.float32),
        dimension_numbers=(((0,), (0,)), ((), ())),
        preferred_element_type=jnp.float32)


def _linearize(table):
    tt = table.T
    return pl.pallas_call(
        _transpose_body,
        grid=(_V // _TBLK,),
        in_specs=[pl.BlockSpec((_D, _TBLK), lambda i: (0, i))],
        out_specs=pl.BlockSpec((_TBLK, _D), lambda i: (i, 0)),
        out_shape=jax.ShapeDtypeStruct((_V, _D), jnp.float32),
    )(tt)


def kernel(inputs, targets, negatives, emb_u, emb_v):
    idxu = inputs.astype(jnp.int32).reshape(-1)
    idxn = negatives.astype(jnp.int32).reshape(-1)
    idxt = targets.astype(jnp.int32)
    pos, neg = _sc_scores()(idxu, idxn, idxt,
                            _linearize(emb_u), _linearize(emb_v))
    res = _finish(pos.reshape(128, 128), neg.reshape(128, 128))
    return res[0, 0]


# speedup vs baseline: 2.0671x; 1.1816x over previous
import functools

import jax
import jax.numpy as jnp
from jax import lax
from jax.experimental import pallas as pl
from jax.experimental.pallas import tpu as pltpu
from jax.experimental.pallas import tpu_sc as plsc

_V, _D = 1_000_000, 64
_B, _C, _K = 16384, 20, 20
_NC, _NS = 2, 16
_NW = _NC * _NS
_BPW = _B // _NW
_NB = 16
_NCHUNK = _BPW // _NB
_ROWS = _NB * _C
_NSTREAM = 4
_SROWS = _ROWS // _NSTREAM
_L = 16


def _sc_body(idxu_hbm, idxn_hbm, idxt_hbm, emb_u_hbm, emb_v_hbm,
             pos_hbm, neg_hbm,
             idx_u, idx_n, idx_t, rows_u, rows_n, rows_t,
             pos_buf, neg_buf, sem):
    wid = lax.axis_index("s") * _NC + lax.axis_index("c")
    base = wid * _BPW
    pltpu.sync_copy(idxu_hbm.at[pl.ds(base * _C, _BPW * _C)], idx_u)
    pltpu.sync_copy(idxn_hbm.at[pl.ds(base * _K, _BPW * _K)], idx_n)
    pltpu.sync_copy(idxt_hbm.at[pl.ds(base, _BPW)], idx_t)

    lanes = lax.iota(jnp.int32, _L)

    def _lane_sum(x):
        for s in (8, 4, 2, 1):
            x = x + x.at[(lanes + s) % _L].get(mode="promise_in_bounds")
        return x

    def chunk_body(ci, _):
        off = ci * _ROWS
        cps = []
        for s in range(_NSTREAM):
            cps.append(pltpu.async_copy(
                emb_u_hbm.at[idx_u.at[pl.ds(off + s * _SROWS, _SROWS)]],
                rows_u.at[pl.ds(s * _SROWS, _SROWS), :], sem))
            cps.append(pltpu.async_copy(
                emb_v_hbm.at[idx_n.at[pl.ds(off + s * _SROWS, _SROWS)]],
                rows_n.at[pl.ds(s * _SROWS, _SROWS), :], sem))
        cps.append(pltpu.async_copy(
            emb_v_hbm.at[idx_t.at[pl.ds(ci * _NB, _NB)]], rows_t, sem))
        for cp in cps:
            cp.wait()

        def b_body(j, carry):
            acc_p, acc_n = carry
            rb = j * _C
            su = [jnp.zeros((_L,), jnp.float32) for _ in range(_D // _L)]
            sn = [jnp.zeros((_L,), jnp.float32) for _ in range(_D // _L)]
            for c in range(_C):
                for blk in range(_D // _L):
                    su[blk] = su[blk] + rows_u[rb + c, pl.ds(blk * _L, _L)]
                    sn[blk] = sn[blk] + rows_n[rb + c, pl.ds(blk * _L, _L)]
            pp = jnp.zeros((_L,), jnp.float32)
            nn = jnp.zeros((_L,), jnp.float32)
            for blk in range(_D // _L):
                t_blk = rows_t[j, pl.ds(blk * _L, _L)]
                pp = pp + t_blk * su[blk]
                nn = nn + sn[blk] * su[blk]
            m = lanes == j
            acc_p = jnp.where(m, _lane_sum(pp) * (1.0 / _C), acc_p)
            acc_n = jnp.where(m, _lane_sum(nn) * (1.0 / _C), acc_n)
            return acc_p, acc_n

        acc_p, acc_n = lax.fori_loop(
            0, _NB, b_body,
            (jnp.zeros((_L,), jnp.float32), jnp.zeros((_L,), jnp.float32)))
        pos_buf[pl.ds(ci * _NB, _NB)] = acc_p
        neg_buf[pl.ds(ci * _NB, _NB)] = acc_n
        return 0

    lax.fori_loop(0, _NCHUNK, chunk_body, 0)
    pltpu.sync_copy(pos_buf, pos_hbm.at[pl.ds(base, _BPW)])
    pltpu.sync_copy(neg_buf, neg_hbm.at[pl.ds(base, _BPW)])


@functools.lru_cache(maxsize=None)
def _sc_scores():
    return functools.partial(
        pl.kernel,
        mesh=plsc.VectorSubcoreMesh(core_axis_name="c", subcore_axis_name="s"),
        compiler_params=pltpu.CompilerParams(use_tc_tiling_on_sc=False),
        out_type=[jax.ShapeDtypeStruct((_B,), jnp.float32),
                  jax.ShapeDtypeStruct((_B,), jnp.float32)],
        scratch_types=[
            pltpu.VMEM((_BPW * _C,), jnp.int32),
            pltpu.VMEM((_BPW * _K,), jnp.int32),
            pltpu.VMEM((_BPW,), jnp.int32),
            pltpu.VMEM((_ROWS, _D), jnp.float32),
            pltpu.VMEM((_ROWS, _D), jnp.float32),
            pltpu.VMEM((_NB, _D), jnp.float32),
            pltpu.VMEM((_BPW,), jnp.float32),
            pltpu.VMEM((_BPW,), jnp.float32),
            pltpu.SemaphoreType.DMA,
        ],
    )(_sc_body)


def _finish_body(pos_ref, neg_ref, out_ref):
    p = pos_ref[...]
    n = neg_ref[...]
    lsp = jnp.minimum(p, 0.0) - jnp.log1p(jnp.exp(-jnp.abs(p)))
    lsn = jnp.minimum(-n, 0.0) - jnp.log1p(jnp.exp(-jnp.abs(n)))
    out_ref[0, 0] = -(jnp.sum(lsp) + jnp.sum(lsn)) / _B


def _finish(pos2d, neg2d):
    return pl.pallas_call(
        _finish_body,
        out_shape=jax.ShapeDtypeStruct((1, 1), jnp.float32),
        out_specs=pl.BlockSpec(memory_space=pltpu.SMEM),
    )(pos2d, neg2d)


_TBLK = 1024
_TGRID = (_V + _TBLK - 1) // _TBLK
_VPAD = _TGRID * _TBLK


def _transpose_body(in_ref, out_ref):
    eye = jnp.eye(_D, dtype=jnp.float32)
    dn = (((0,), (0,)), ((), ()))
    out_ref[:, 0:_D] = lax.dot_general(
        in_ref[:, 0:_TBLK // 2], eye, dn, preferred_element_type=jnp.float32)
    out_ref[:, _D:2 * _D] = lax.dot_general(
        in_ref[:, _TBLK // 2:_TBLK], eye, dn,
        preferred_element_type=jnp.float32)


def _linearize(table):
    tt = table.T
    out = pl.pallas_call(
        _transpose_body,
        grid=(_TGRID,),
        in_specs=[pl.BlockSpec((_D, _TBLK), lambda i: (0, i))],
        out_specs=pl.BlockSpec((_TBLK // 2, 2 * _D), lambda i: (i, 0)),
        out_shape=jax.ShapeDtypeStruct((_VPAD // 2, 2 * _D), jnp.float32),
    )(tt)
    return out.reshape(_VPAD, _D)


def _remap(v):
    return (v & ~1023) | ((v & 511) << 1) | ((v >> 9) & 1)


def kernel(inputs, targets, negatives, emb_u, emb_v):
    idxu = _remap(inputs.astype(jnp.int32)).reshape(-1)
    idxn = _remap(negatives.astype(jnp.int32)).reshape(-1)
    idxt = _remap(targets.astype(jnp.int32))
    pos, neg = _sc_scores()(idxu, idxn, idxt,
                            _linearize(emb_u), _linearize(emb_v))
    res = _finish(pos.reshape(128, 128), neg.reshape(128, 128))
    return res[0, 0]


# speedup vs baseline: 2.4642x; 1.1921x over previous
import functools

import jax
import jax.numpy as jnp
from jax import lax
from jax.experimental import pallas as pl
from jax.experimental.pallas import tpu as pltpu
from jax.experimental.pallas import tpu_sc as plsc

_V, _D = 1_000_000, 64
_B, _C, _K = 16384, 20, 20
_NC, _NS = 2, 16
_NW = _NC * _NS
_BPW = _B // _NW
_NB = 16
_NCHUNK = _BPW // _NB
_ROWS = _NB * _C
_NSTREAM = 4
_SROWS = _ROWS // _NSTREAM
_L = 16


def _sc_body(idxu_hbm, idxn_hbm, idxt_hbm, emb_u_hbm, emb_v_hbm,
             pos_hbm, neg_hbm,
             idx_u, idx_n, idx_t, rows_u, rows_n, rows_t,
             pos_buf, neg_buf, sem):
    wid = lax.axis_index("s") * _NC + lax.axis_index("c")
    base = wid * _BPW
    pltpu.sync_copy(idxu_hbm.at[pl.ds(base * _C, _BPW * _C)], idx_u)
    pltpu.sync_copy(idxn_hbm.at[pl.ds(base * _K, _BPW * _K)], idx_n)
    pltpu.sync_copy(idxt_hbm.at[pl.ds(base, _BPW)], idx_t)

    lanes = lax.iota(jnp.int32, _L)

    def _lane_sum(x):
        for s in (8, 4, 2, 1):
            x = x + x.at[(lanes + s) % _L].get(mode="promise_in_bounds")
        return x

    def chunk_body(ci, _):
        off = ci * _ROWS
        cps = []
        for s in range(_NSTREAM):
            cps.append(pltpu.async_copy(
                emb_u_hbm.at[idx_u.at[pl.ds(off + s * _SROWS, _SROWS)]],
                rows_u.at[pl.ds(s * _SROWS, _SROWS), :], sem))
            cps.append(pltpu.async_copy(
                emb_v_hbm.at[idx_n.at[pl.ds(off + s * _SROWS, _SROWS)]],
                rows_n.at[pl.ds(s * _SROWS, _SROWS), :], sem))
        cps.append(pltpu.async_copy(
            emb_v_hbm.at[idx_t.at[pl.ds(ci * _NB, _NB)]], rows_t, sem))
        for cp in cps:
            cp.wait()

        def b_body(j, carry):
            acc_p, acc_n = carry
            rb = j * _C
            su = [jnp.zeros((_L,), jnp.float32) for _ in range(_D // _L)]
            sn = [jnp.zeros((_L,), jnp.float32) for _ in range(_D // _L)]
            for c in range(_C):
                for blk in range(_D // _L):
                    su[blk] = su[blk] + rows_u[rb + c, pl.ds(blk * _L, _L)]
                    sn[blk] = sn[blk] + rows_n[rb + c, pl.ds(blk * _L, _L)]
            pp = jnp.zeros((_L,), jnp.float32)
            nn = jnp.zeros((_L,), jnp.float32)
            for blk in range(_D // _L):
                t_blk = rows_t[j, pl.ds(blk * _L, _L)]
                pp = pp + t_blk * su[blk]
                nn = nn + sn[blk] * su[blk]
            m = lanes == j
            acc_p = jnp.where(m, _lane_sum(pp) * (1.0 / _C), acc_p)
            acc_n = jnp.where(m, _lane_sum(nn) * (1.0 / _C), acc_n)
            return acc_p, acc_n

        acc_p, acc_n = lax.fori_loop(
            0, _NB, b_body,
            (jnp.zeros((_L,), jnp.float32), jnp.zeros((_L,), jnp.float32)))
        pos_buf[pl.ds(ci * _NB, _NB)] = acc_p
        neg_buf[pl.ds(ci * _NB, _NB)] = acc_n
        return 0

    lax.fori_loop(0, _NCHUNK, chunk_body, 0)
    pltpu.sync_copy(pos_buf, pos_hbm.at[pl.ds(base, _BPW)])
    pltpu.sync_copy(neg_buf, neg_hbm.at[pl.ds(base, _BPW)])


@functools.lru_cache(maxsize=None)
def _sc_scores():
    return functools.partial(
        pl.kernel,
        mesh=plsc.VectorSubcoreMesh(core_axis_name="c", subcore_axis_name="s"),
        compiler_params=pltpu.CompilerParams(use_tc_tiling_on_sc=False),
        out_type=[jax.ShapeDtypeStruct((_B,), jnp.float32),
                  jax.ShapeDtypeStruct((_B,), jnp.float32)],
        scratch_types=[
            pltpu.VMEM((_BPW * _C,), jnp.int32),
            pltpu.VMEM((_BPW * _K,), jnp.int32),
            pltpu.VMEM((_BPW,), jnp.int32),
            pltpu.VMEM((_ROWS, _D), jnp.float32),
            pltpu.VMEM((_ROWS, _D), jnp.float32),
            pltpu.VMEM((_NB, _D), jnp.float32),
            pltpu.VMEM((_BPW,), jnp.float32),
            pltpu.VMEM((_BPW,), jnp.float32),
            pltpu.SemaphoreType.DMA,
        ],
    )(_sc_body)


def _finish_body(pos_ref, neg_ref, out_ref):
    p = pos_ref[...]
    n = neg_ref[...]
    lsp = jnp.minimum(p, 0.0) - jnp.log1p(jnp.exp(-jnp.abs(p)))
    lsn = jnp.minimum(-n, 0.0) - jnp.log1p(jnp.exp(-jnp.abs(n)))
    out_ref[0, 0] = -(jnp.sum(lsp) + jnp.sum(lsn)) / _B


def _finish(pos2d, neg2d):
    return pl.pallas_call(
        _finish_body,
        out_shape=jax.ShapeDtypeStruct((1, 1), jnp.float32),
        out_specs=pl.BlockSpec(memory_space=pltpu.SMEM),
    )(pos2d, neg2d)


_TBLK = 1024
_TGRID = (_V + _TBLK - 1) // _TBLK
_VPAD = _TGRID * _TBLK


def _transpose_body(in_ref, out_ref):
    eye = jnp.eye(_D, dtype=jnp.float32)
    dn = (((0,), (0,)), ((), ()))
    out_ref[:, 0:_D] = lax.dot_general(
        in_ref[:, 0:_TBLK // 2], eye, dn, preferred_element_type=jnp.float32)
    out_ref[:, _D:2 * _D] = lax.dot_general(
        in_ref[:, _TBLK // 2:_TBLK], eye, dn,
        preferred_element_type=jnp.float32)


def _linearize(table):
    tt = table.T
    out = pl.pallas_call(
        _transpose_body,
        grid=(_TGRID,),
        in_specs=[pl.BlockSpec((_D, _TBLK), lambda i: (0, i))],
        out_specs=pl.BlockSpec((_TBLK // 2, 2 * _D), lambda i: (i, 0)),
        out_shape=jax.ShapeDtypeStruct((_VPAD // 2, 2 * _D), jnp.float32),
    )(tt)
    return out.reshape(_VPAD, _D)


def _remap(v):
    return (v & ~1023) | ((v & 511) << 1) | ((v >> 9) & 1)


def kernel(inputs, targets, negatives, emb_u, emb_v):
    idxu = inputs.astype(jnp.int32).reshape(-1)
    idxn = _remap(negatives.astype(jnp.int32)).reshape(-1)
    idxt = _remap(targets.astype(jnp.int32))
    pos, neg = _sc_scores()(idxu, idxn, idxt, emb_u, _linearize(emb_v))
    res = _finish(pos.reshape(128, 128), neg.reshape(128, 128))
    return res[0, 0]
